# baseline (speedup 1.0000x reference)
import jax
import jax.numpy as jnp
from jax.experimental import pallas as pl
from jax.experimental.pallas import tpu as pltpu

N = 10000
DEG = 16
D = 256
BN = 200
G = N // BN


def _ce_copy(ce_hbm, ce_vmem, sems, block, slot):
    return pltpu.make_async_copy(
        ce_hbm.at[pl.ds(block * BN, BN), 0, :], ce_vmem.at[slot], sems.at[slot])


def _body(ce_hbm, al_ref, msg_ref, out_ref, ce_vmem, sems):
    i = pl.program_id(0)
    slot = jax.lax.rem(i, 2)

    @pl.when(i == 0)
    def _():
        _ce_copy(ce_hbm, ce_vmem, sems, 0, 0).start()

    @pl.when(i + 1 < G)
    def _():
        _ce_copy(ce_hbm, ce_vmem, sems, i + 1, jax.lax.rem(i + 1, 2)).start()

    al = al_ref[...]
    m = msg_ref[...]
    acc = jnp.sum(al[:, :, None] * m, axis=1)
    _ce_copy(ce_hbm, ce_vmem, sems, i, slot).wait()
    out_ref[...] = acc + ce_vmem[slot]


def kernel(curr_emb, alpha, msg):
    al2 = jnp.squeeze(alpha, -1)
    return pl.pallas_call(
        _body,
        grid=(G,),
        in_specs=[
            pl.BlockSpec(memory_space=pl.ANY),
            pl.BlockSpec((BN, DEG), lambda i: (i, 0)),
            pl.BlockSpec((BN, DEG, D), lambda i: (i, 0, 0)),
        ],
        out_specs=pl.BlockSpec((BN, D), lambda i: (i, 0)),
        out_shape=jax.ShapeDtypeStruct((N, D), jnp.float32),
        scratch_shapes=[
            pltpu.VMEM((2, BN, D), jnp.float32),
            pltpu.SemaphoreType.DMA((2,)),
        ],
    )(curr_emb, al2, msg)


# speedup vs baseline: 1.2185x; 1.2185x over previous
import jax
import jax.numpy as jnp
from jax.experimental import pallas as pl
from jax.experimental.pallas import tpu as pltpu

N = 10000
DEG = 16
D = 256
BN = 1000
G = N // BN


def _ce_copy(ce_hbm, ce_vmem, sems, block, slot):
    return pltpu.make_async_copy(
        ce_hbm.at[pl.ds(block * BN, BN), 0, :], ce_vmem.at[slot], sems.at[slot])


def _body(ce_hbm, al_ref, msg_ref, out_ref, ce_vmem, sems):
    i = pl.program_id(0)
    slot = jax.lax.rem(i, 2)

    @pl.when(i == 0)
    def _():
        _ce_copy(ce_hbm, ce_vmem, sems, 0, 0).start()

    @pl.when(i + 1 < G)
    def _():
        _ce_copy(ce_hbm, ce_vmem, sems, i + 1, jax.lax.rem(i + 1, 2)).start()

    al = al_ref[...]
    m = msg_ref[...]
    acc = jnp.sum(al[:, :, None] * m, axis=1)
    _ce_copy(ce_hbm, ce_vmem, sems, i, slot).wait()
    out_ref[...] = acc + ce_vmem[slot]


def kernel(curr_emb, alpha, msg):
    al2 = jnp.squeeze(alpha, -1)
    return pl.pallas_call(
        _body,
        grid=(G,),
        in_specs=[
            pl.BlockSpec(memory_space=pl.ANY),
            pl.BlockSpec((BN, DEG), lambda i: (i, 0)),
            pl.BlockSpec((BN, DEG, D), lambda i: (i, 0, 0)),
        ],
        out_specs=pl.BlockSpec((BN, D), lambda i: (i, 0)),
        out_shape=jax.ShapeDtypeStruct((N, D), jnp.float32),
        scratch_shapes=[
            pltpu.VMEM((2, BN, D), jnp.float32),
            pltpu.SemaphoreType.DMA((2,)),
        ],
    )(curr_emb, al2, msg)
